# Initial kernel scaffold; baseline (speedup 1.0000x reference)
import jax
import jax.numpy as jnp
from jax.experimental import pallas as pl


def kernel(indices, table):
    raise NotImplementedError("write your pallas kernel here")


# speedup vs baseline: 3.3236x; 3.3236x over previous
import functools

import jax
import jax.numpy as jnp
from jax import lax
from jax.experimental import pallas as pl
from jax.experimental.pallas import tpu as pltpu
from jax.experimental.pallas import tpu_sc as plsc

EMBED_DIM = 64

_info = plsc.get_sparse_core_info()
_NC, _NS = _info.num_cores, _info.num_subcores
_NW = _NC * _NS


@functools.partial(jax.jit, static_argnames=("total", "chunk"))
def _lookup(idx_flat, table, *, total, chunk):
    per_w = total // _NW
    n_chunks = per_w // chunk
    mesh = plsc.VectorSubcoreMesh(core_axis_name="c", subcore_axis_name="s")

    @functools.partial(
        pl.kernel,
        mesh=mesh,
        out_type=jax.ShapeDtypeStruct((total, EMBED_DIM), jnp.float32),
        scratch_types=[
            pltpu.VMEM((chunk,), jnp.int32),
            pltpu.VMEM((chunk, EMBED_DIM), jnp.float32),
            pltpu.SemaphoreType.DMA,
        ],
        compiler_params=pltpu.CompilerParams(use_tc_tiling_on_sc=False),
    )
    def k(idx_hbm, table_hbm, out_hbm, idx_v, rows_v, sem):
        wid = lax.axis_index("s") * _NC + lax.axis_index("c")
        base = wid * per_w

        def body(i, _):
            off = base + i * chunk
            pltpu.sync_copy(idx_hbm.at[pl.ds(off, chunk)], idx_v)
            pltpu.async_copy(table_hbm.at[idx_v], rows_v, sem).wait()
            pltpu.sync_copy(rows_v, out_hbm.at[pl.ds(off, chunk)])
            return ()

        lax.fori_loop(0, n_chunks, body, ())

    return k(idx_flat, table)


def kernel(indices, table):
    B, L = indices.shape
    total = B * L
    idx_flat = indices.reshape(total).astype(jnp.int32)
    out = _lookup(idx_flat, table, total=total, chunk=512)
    return out.reshape(B, L, EMBED_DIM)


# speedup vs baseline: 3.3259x; 1.0007x over previous
import functools

import jax
import jax.numpy as jnp
from jax import lax
from jax.experimental import pallas as pl
from jax.experimental.pallas import tpu as pltpu
from jax.experimental.pallas import tpu_sc as plsc

EMBED_DIM = 64

_info = plsc.get_sparse_core_info()
_NC, _NS = _info.num_cores, _info.num_subcores
_NW = _NC * _NS


@functools.partial(jax.jit, static_argnames=("total", "chunk", "nbuf"))
def _lookup(idx2d, table, *, total, chunk, nbuf):
    per_w = total // _NW
    gsize = chunk * nbuf
    ngroups = per_w // gsize
    assert ngroups * gsize == per_w
    chunks_per_w = per_w // chunk
    mesh = plsc.VectorSubcoreMesh(core_axis_name="c", subcore_axis_name="s")

    @functools.partial(
        pl.kernel,
        mesh=mesh,
        out_type=jax.ShapeDtypeStruct((total, EMBED_DIM), jnp.float32),
        scratch_types=[
            pltpu.VMEM((2 * nbuf, chunk), jnp.int32),
            pltpu.VMEM((nbuf, chunk, EMBED_DIM), jnp.float32),
            pltpu.SemaphoreType.DMA,
            pltpu.SemaphoreType.DMA((nbuf,)),
            pltpu.SemaphoreType.DMA((nbuf,)),
        ],
        compiler_params=pltpu.CompilerParams(use_tc_tiling_on_sc=False),
    )
    def k(idx_hbm, table_hbm, out_hbm, idx_v, rows_v, isem, gsem, wsem):
        wid = lax.axis_index("s") * _NC + lax.axis_index("c")
        base = wid * per_w
        crow0 = wid * chunks_per_w

        pltpu.async_copy(
            idx_hbm.at[pl.ds(crow0, nbuf)], idx_v.at[pl.ds(0, nbuf)], isem
        )

        def group(g, _):
            par = g % 2
            pltpu.make_async_copy(
                idx_hbm.at[pl.ds(crow0, nbuf)], idx_v.at[pl.ds(par * nbuf, nbuf)], isem
            ).wait()

            @pl.when(g < ngroups - 1)
            def _prefetch():
                row_n = crow0 + (g + 1) * nbuf
                pltpu.async_copy(
                    idx_hbm.at[pl.ds(row_n, nbuf)],
                    idx_v.at[pl.ds((1 - par) * nbuf, nbuf)],
                    isem,
                )

            handles = []
            for b in range(nbuf):
                @pl.when(g > 0)
                def _drain(b=b):
                    pltpu.make_async_copy(
                        rows_v.at[b], out_hbm.at[pl.ds(base, chunk)], wsem.at[b]
                    ).wait()

                handles.append(
                    pltpu.async_copy(
                        table_hbm.at[idx_v.at[par * nbuf + b]],
                        rows_v.at[b],
                        gsem.at[b],
                    )
                )
            for b in range(nbuf):
                handles[b].wait()
                off = base + g * gsize + b * chunk
                pltpu.async_copy(
                    rows_v.at[b], out_hbm.at[pl.ds(off, chunk)], wsem.at[b]
                )
            return ()

        lax.fori_loop(0, ngroups, group, ())

        for b in range(nbuf):
            pltpu.make_async_copy(
                rows_v.at[b], out_hbm.at[pl.ds(base, chunk)], wsem.at[b]
            ).wait()

    return k(idx2d, table)


def kernel(indices, table):
    B, L = indices.shape
    total = B * L
    chunk = 512
    idx2d = indices.reshape(total // chunk, chunk).astype(jnp.int32)
    out = _lookup(idx2d, table, total=total, chunk=chunk, nbuf=2)
    return out.reshape(B, L, EMBED_DIM)


# speedup vs baseline: 5.4361x; 1.6345x over previous
import functools

import jax
import jax.numpy as jnp
from jax import lax
from jax.experimental import pallas as pl
from jax.experimental.pallas import tpu as pltpu
from jax.experimental.pallas import tpu_sc as plsc

EMBED_DIM = 64

_info = plsc.get_sparse_core_info()
_NC, _NS = _info.num_cores, _info.num_subcores
_NW = _NC * _NS


@functools.partial(jax.jit, static_argnames=("total", "chunk", "nbuf"))
def _lookup(idx2d, table, *, total, chunk, nbuf):
    per_w = total // _NW
    gsize = chunk * nbuf
    ngroups = per_w // gsize
    assert ngroups * gsize == per_w
    chunks_per_w = per_w // chunk
    mesh = plsc.VectorSubcoreMesh(core_axis_name="c", subcore_axis_name="s")

    @functools.partial(
        pl.kernel,
        mesh=mesh,
        out_type=jax.ShapeDtypeStruct((total, EMBED_DIM), jnp.float32),
        scratch_types=[
            pltpu.VMEM((2 * nbuf, chunk), jnp.int32),
            pltpu.VMEM((nbuf, chunk, EMBED_DIM), jnp.float32),
            pltpu.VMEM_SHARED((128, EMBED_DIM), jnp.float32),
            pltpu.SemaphoreType.DMA,
            pltpu.SemaphoreType.DMA((nbuf,)),
            pltpu.SemaphoreType.DMA((nbuf,)),
        ],
        compiler_params=pltpu.CompilerParams(use_tc_tiling_on_sc=False),
    )
    def k(idx_hbm, table_hbm, out_hbm, idx_v, rows_v, table_sh, isem, gsem, wsem):
        sid = lax.axis_index("s")
        wid = sid * _NC + lax.axis_index("c")
        base = wid * per_w
        crow0 = wid * chunks_per_w

        @pl.when(sid == 0)
        def _stage_table():
            pltpu.sync_copy(table_hbm, table_sh)

        plsc.subcore_barrier()

        pltpu.async_copy(
            idx_hbm.at[pl.ds(crow0, nbuf)], idx_v.at[pl.ds(0, nbuf)], isem
        )

        def group(g, _):
            par = g % 2
            pltpu.make_async_copy(
                idx_hbm.at[pl.ds(crow0, nbuf)], idx_v.at[pl.ds(par * nbuf, nbuf)], isem
            ).wait()

            @pl.when(g < ngroups - 1)
            def _prefetch():
                row_n = crow0 + (g + 1) * nbuf
                pltpu.async_copy(
                    idx_hbm.at[pl.ds(row_n, nbuf)],
                    idx_v.at[pl.ds((1 - par) * nbuf, nbuf)],
                    isem,
                )

            handles = []
            for b in range(nbuf):
                @pl.when(g > 0)
                def _drain(b=b):
                    pltpu.make_async_copy(
                        rows_v.at[b], out_hbm.at[pl.ds(base, chunk)], wsem.at[b]
                    ).wait()

                handles.append(
                    pltpu.async_copy(
                        table_sh.at[idx_v.at[par * nbuf + b]],
                        rows_v.at[b],
                        gsem.at[b],
                    )
                )
            for b in range(nbuf):
                handles[b].wait()
                off = base + g * gsize + b * chunk
                pltpu.async_copy(
                    rows_v.at[b], out_hbm.at[pl.ds(off, chunk)], wsem.at[b]
                )
            return ()

        lax.fori_loop(0, ngroups, group, ())

        for b in range(nbuf):
            pltpu.make_async_copy(
                rows_v.at[b], out_hbm.at[pl.ds(base, chunk)], wsem.at[b]
            ).wait()

    return k(idx2d, table)


def kernel(indices, table):
    B, L = indices.shape
    total = B * L
    chunk = 512
    idx2d = indices.reshape(total // chunk, chunk).astype(jnp.int32)
    out = _lookup(idx2d, table, total=total, chunk=chunk, nbuf=2)
    return out.reshape(B, L, EMBED_DIM)


# speedup vs baseline: 5.8134x; 1.0694x over previous
import functools

import jax
import jax.numpy as jnp
from jax import lax
from jax.experimental import pallas as pl
from jax.experimental.pallas import tpu as pltpu
from jax.experimental.pallas import tpu_sc as plsc

EMBED_DIM = 64

_info = plsc.get_sparse_core_info()
_NC, _NS = _info.num_cores, _info.num_subcores
_NW = _NC * _NS


@functools.partial(jax.jit, static_argnames=("batch", "length"))
def _lookup(idx2d, table, *, batch, length):
    chunk_b = 2
    chunk = chunk_b * length
    per_w_b = batch // _NW
    nchunks = per_w_b // chunk_b
    mesh = plsc.VectorSubcoreMesh(core_axis_name="c", subcore_axis_name="s")

    @functools.partial(
        pl.kernel,
        mesh=mesh,
        out_type=jax.ShapeDtypeStruct((batch, length, EMBED_DIM), jnp.float32),
        scratch_types=[
            pltpu.VMEM((2, chunk_b, length), jnp.int32),
            pltpu.VMEM((2, chunk_b, length, EMBED_DIM), jnp.float32),
            pltpu.VMEM_SHARED((128, EMBED_DIM), jnp.float32),
            pltpu.SemaphoreType.DMA,
            pltpu.SemaphoreType.DMA,
            pltpu.SemaphoreType.DMA,
            pltpu.SemaphoreType.DMA,
            pltpu.SemaphoreType.DMA,
            pltpu.SemaphoreType.DMA,
        ],
        compiler_params=pltpu.CompilerParams(use_tc_tiling_on_sc=False),
    )
    def k(idx_hbm, table_hbm, out_hbm, idx_v, rows_v, table_sh,
          isem0, isem1, gsem0, gsem1, wsem0, wsem1):
        sid = lax.axis_index("s")
        wid = sid * _NC + lax.axis_index("c")
        b_base = wid * per_w_b
        isems = (isem0, isem1)
        gsems = (gsem0, gsem1)
        wsems = (wsem0, wsem1)

        @pl.when(sid == 0)
        def _stage_table():
            pltpu.sync_copy(table_hbm, table_sh)

        plsc.subcore_barrier()

        for par in range(2):
            pltpu.async_copy(
                idx_hbm.at[pl.ds(b_base + par * chunk_b, chunk_b)],
                idx_v.at[par],
                isems[par],
            )

        def super_chunk(c2, _):
            for par in range(2):
                c = 2 * c2 + par
                pltpu.make_async_copy(
                    idx_hbm.at[pl.ds(b_base, chunk_b)], idx_v.at[par],
                    isems[par],
                ).wait()
                @pl.when(c2 > 0)
                def _drain():
                    pltpu.make_async_copy(
                        rows_v.at[par], out_hbm.at[pl.ds(b_base, chunk_b)],
                        wsems[par],
                    ).wait()

                handles = [
                    pltpu.async_copy(
                        table_sh.at[idx_v.at[par, cb]],
                        rows_v.at[par, cb],
                        gsems[par],
                    )
                    for cb in range(chunk_b)
                ]
                for h in handles:
                    h.wait()

                @pl.when(c < nchunks - 2)
                def _prefetch():
                    pltpu.async_copy(
                        idx_hbm.at[pl.ds(b_base + (c + 2) * chunk_b, chunk_b)],
                        idx_v.at[par],
                        isems[par],
                    )

                pltpu.async_copy(
                    rows_v.at[par],
                    out_hbm.at[pl.ds(b_base + c * chunk_b, chunk_b)],
                    wsems[par],
                )
            return ()

        lax.fori_loop(0, nchunks // 2, super_chunk, ())

        for par in range(2):
            pltpu.make_async_copy(
                rows_v.at[par], out_hbm.at[pl.ds(b_base, chunk_b)], wsems[par]
            ).wait()

    return k(idx2d, table)


def kernel(indices, table):
    B, L = indices.shape
    return _lookup(indices.astype(jnp.int32), table, batch=B, length=L)
